# Initial kernel scaffold; baseline (speedup 1.0000x reference)
import jax
import jax.numpy as jnp
from jax.experimental import pallas as pl


def kernel(c1_w, c1_b, c2_w, c2_b, f1_w, f1_b, f2_w, f2_b, f3_w, f3_b, x):
    raise NotImplementedError("write your pallas kernel here")


# speedup vs baseline: 145.1114x; 145.1114x over previous
import numpy as np

import jax
import jax.numpy as jnp
from jax.experimental import pallas as pl
from jax.experimental.pallas import tpu as pltpu

B_TILE = 256
N1 = 864
N1P = 896
N2 = 256
K1 = 784
K2P = 896


def _round_up(x, m):
    return (x + m - 1) // m * m


def _sel(n_in, n_out, k, a):
    r = np.arange(n_in)[:, None, None]
    i = np.arange(n_out)[None, :, None]
    d = np.arange(k)[None, None, :]
    return (r == 2 * i + a + d).astype(np.float32)


def _conv1_big_w(c1_w):
    w1 = c1_w[:25, :6].reshape(5, 5, 6)
    cols = []
    for a in (0, 1):
        ri = _sel(28, 12, 5, a)
        t = jnp.einsum('rid,dDo->riDo', ri, w1)
        for e in (0, 1):
            cj = _sel(28, 12, 5, e)
            w = jnp.einsum('riDo,cjD->rcijo', t, cj)
            w = w.reshape(K1, N1)
            cols.append(jnp.pad(w, ((0, 0), (0, N1P - N1))))
    return jnp.concatenate(cols, axis=1)


def _conv2_big_w(c2_w):
    w2 = c2_w[:150, :16].reshape(6, 5, 5, 16)
    cols = []
    for a in (0, 1):
        ri = _sel(12, 4, 5, a)
        t = jnp.einsum('rid,CdDo->riCDo', ri, w2)
        for e in (0, 1):
            cj = _sel(12, 4, 5, e)
            w = jnp.einsum('riCDo,cjD->rcCijo', t, cj)
            w = w.reshape(N1, N2)
            cols.append(jnp.pad(w, ((0, K2P - N1), (0, 0))))
    return jnp.concatenate(cols, axis=1)


def _net_kernel(x_ref, w1_ref, b1_ref, w2_ref, b2_ref,
                f1w_ref, f1b_ref, f2w_ref, f2b_ref, f3w_ref, f3b_ref,
                o_ref, z1_s, f1_s, z2_s):
    f32 = jnp.float32
    z1_s[...] = jnp.dot(x_ref[...], w1_ref[...], preferred_element_type=f32)
    y = jnp.maximum(
        jnp.maximum(z1_s[:, 0 * N1P:1 * N1P], z1_s[:, 1 * N1P:2 * N1P]),
        jnp.maximum(z1_s[:, 2 * N1P:3 * N1P], z1_s[:, 3 * N1P:4 * N1P]))
    f1_s[...] = jnp.maximum(y + b1_ref[...], 0.0)
    z2_s[...] = jnp.dot(f1_s[...], w2_ref[...], preferred_element_type=f32)
    xf = jnp.maximum(
        jnp.maximum(z2_s[:, 0 * N2:1 * N2], z2_s[:, 1 * N2:2 * N2]),
        jnp.maximum(z2_s[:, 2 * N2:3 * N2], z2_s[:, 3 * N2:4 * N2]))
    xf = jnp.maximum(xf + b2_ref[...], 0.0)
    h = jnp.dot(xf, f1w_ref[...], preferred_element_type=f32)
    h = jnp.maximum(h + f1b_ref[...], 0.0)
    h = jnp.dot(h, f2w_ref[...], preferred_element_type=f32)
    h = jnp.maximum(h + f2b_ref[...], 0.0)
    yy = jnp.dot(h, f3w_ref[...], preferred_element_type=f32) + f3b_ref[...]
    m = jnp.max(yy, axis=-1, keepdims=True)
    e = jnp.exp(yy - m)
    s = jnp.sum(e, axis=-1, keepdims=True)
    o_ref[...] = (yy - m - jnp.log(s)).astype(o_ref.dtype)


def kernel(c1_w, c1_b, c2_w, c2_b, f1_w, f1_b, f2_w, f2_b, f3_w, f3_b, x):
    bsz = x.shape[0]
    xb = x.astype(jnp.float32).reshape(bsz, K1)
    b_pad = _round_up(bsz, B_TILE)
    if b_pad != bsz:
        xb = jnp.pad(xb, ((0, b_pad - bsz), (0, 0)))

    w1 = _conv1_big_w(c1_w)
    b1 = jnp.pad(jnp.tile(c1_b[:1, :6], (1, 144)), ((0, 0), (0, N1P - N1)))
    w2 = _conv2_big_w(c2_w)
    b2 = jnp.tile(c2_b[:1, :16], (1, 16))

    out = pl.pallas_call(
        _net_kernel,
        out_shape=jax.ShapeDtypeStruct((b_pad, 128), jnp.float32),
        grid=(b_pad // B_TILE,),
        in_specs=[
            pl.BlockSpec((B_TILE, K1), lambda i: (i, 0)),
            pl.BlockSpec((K1, 4 * N1P), lambda i: (0, 0)),
            pl.BlockSpec((1, N1P), lambda i: (0, 0)),
            pl.BlockSpec((K2P, 4 * N2), lambda i: (0, 0)),
            pl.BlockSpec((1, N2), lambda i: (0, 0)),
            pl.BlockSpec((256, 128), lambda i: (0, 0)),
            pl.BlockSpec((1, 128), lambda i: (0, 0)),
            pl.BlockSpec((128, 128), lambda i: (0, 0)),
            pl.BlockSpec((1, 128), lambda i: (0, 0)),
            pl.BlockSpec((128, 128), lambda i: (0, 0)),
            pl.BlockSpec((1, 128), lambda i: (0, 0)),
        ],
        out_specs=pl.BlockSpec((B_TILE, 128), lambda i: (i, 0)),
        scratch_shapes=[
            pltpu.VMEM((B_TILE, 4 * N1P), jnp.float32),
            pltpu.VMEM((B_TILE, N1P), jnp.float32),
            pltpu.VMEM((B_TILE, 4 * N2), jnp.float32),
        ],
        compiler_params=pltpu.CompilerParams(
            dimension_semantics=("parallel",)),
    )(xb, w1, b1, w2, b2, f1_w, f1_b, f2_w, f2_b, f3_w, f3_b)
    return out[:bsz, :2]
